# baseline (speedup 1.0000x reference)
import functools

import jax
import jax.numpy as jnp
from jax import lax
from jax.experimental import pallas as pl
from jax.experimental.pallas import tpu as pltpu
from jax.experimental.pallas import tpu_sc as plsc

TOKENS_PER_BLOCK = 2048
NUM_EXPERTS = 64


def _logits_block(x_ref, w_ref, b_ref, out_ref):
    xb = x_ref[...]
    wb = w_ref[...]
    logits = jax.lax.dot_general(
        xb, wb, (((1,), (1,)), ((), ())),
        preferred_element_type=jnp.float32)
    out_ref[...] = (logits + b_ref[...]).T


def _tc_logits(xt, W, b):
    n, d = xt.shape
    t = TOKENS_PER_BLOCK
    return pl.pallas_call(
        _logits_block,
        grid=(n // t,),
        in_specs=[
            pl.BlockSpec((t, d), lambda i: (i, 0)),
            pl.BlockSpec((NUM_EXPERTS, d), lambda i: (0, 0)),
            pl.BlockSpec((1, NUM_EXPERTS), lambda i: (0, 0)),
        ],
        out_specs=pl.BlockSpec((NUM_EXPERTS, t), lambda i: (0, i)),
        out_shape=jax.ShapeDtypeStruct((NUM_EXPERTS, n), jnp.float32),
        compiler_params=pltpu.CompilerParams(
            dimension_semantics=("parallel",),
        ),
    )(xt, W, b.reshape(1, NUM_EXPERTS))


def _make_sc_topk(n):
    info = plsc.get_sparse_core_info()
    nc, ns, nl = info.num_cores, info.num_subcores, info.num_lanes
    nw = nc * ns
    tpw = n // nw
    ngroups = tpw // nl
    mesh = plsc.VectorSubcoreMesh(core_axis_name="c", subcore_axis_name="s")

    @functools.partial(
        pl.kernel, mesh=mesh,
        out_type=[
            jax.ShapeDtypeStruct((2, n), jnp.float32),
            jax.ShapeDtypeStruct((2, n), jnp.int32),
        ],
        scratch_types=[
            pltpu.VMEM((NUM_EXPERTS, tpw), jnp.float32),
            pltpu.VMEM((2, tpw), jnp.float32),
            pltpu.VMEM((2, tpw), jnp.int32),
        ],
    )
    def sc_topk(logits_hbm, w_out, i_out, buf, wbuf, ibuf):
        wid = lax.axis_index("s") * nc + lax.axis_index("c")
        base = wid * tpw
        pltpu.sync_copy(logits_hbm.at[:, pl.ds(base, tpw)], buf)

        def group(g, carry):
            sl = pl.ds(g * nl, nl)

            def col(e):
                return buf[e, sl]

            m1 = col(0)
            i1 = jnp.zeros((nl,), jnp.int32)
            m2 = jnp.full((nl,), -jnp.inf, jnp.float32)
            i2 = jnp.zeros((nl,), jnp.int32)
            for e in range(1, NUM_EXPERTS):
                v = col(e)
                ev = jnp.full((nl,), e, jnp.int32)
                gt1 = v > m1
                gt2 = v > m2
                m2 = jnp.where(gt1, m1, jnp.where(gt2, v, m2))
                i2 = jnp.where(gt1, i1, jnp.where(gt2, ev, i2))
                m1 = jnp.where(gt1, v, m1)
                i1 = jnp.where(gt1, ev, i1)

            s = jnp.exp(m2 - m1)
            den = 1.0 + s
            wbuf[0, sl] = 1.0 / den
            wbuf[1, sl] = s / den
            ibuf[0, sl] = i1
            ibuf[1, sl] = i2
            return carry

        lax.fori_loop(0, ngroups, group, 0)

        pltpu.sync_copy(wbuf, w_out.at[:, pl.ds(base, tpw)])
        pltpu.sync_copy(ibuf, i_out.at[:, pl.ds(base, tpw)])

    return sc_topk


@functools.partial(jax.jit, static_argnames=())
def kernel(x, W, b):
    d = x.shape[-1]
    xt = x.reshape(-1, d)
    n = xt.shape[0]
    logits = _tc_logits(xt, W, b)
    wt, it = _make_sc_topk(n)(logits)
    return (wt.T, it.T)


# speedup vs baseline: 1.0495x; 1.0495x over previous
import functools

import jax
import jax.numpy as jnp
from jax.experimental import pallas as pl
from jax.experimental.pallas import tpu as pltpu

CHUNK = 1024
NBUF = 5
NUM_EXPERTS = 64


def _top2_softmax(logits, t):
    eidx = jax.lax.broadcasted_iota(jnp.int32, (t, NUM_EXPERTS), 1)
    m1 = jnp.max(logits, axis=1, keepdims=True)
    i1 = jnp.argmax(logits, axis=1).reshape(t, 1).astype(jnp.int32)
    masked = jnp.where(eidx == i1, -jnp.inf, logits)
    m2 = jnp.max(masked, axis=1, keepdims=True)
    i2 = jnp.argmax(masked, axis=1).reshape(t, 1).astype(jnp.int32)
    s = jnp.exp(m2 - m1)
    w1 = 1.0 / (1.0 + s)
    w2 = s / (1.0 + s)
    return (jnp.concatenate([w1, w2], axis=1),
            jnp.concatenate([i1, i2], axis=1))


def _router_body(x_hbm, w_ref, b_ref, w_out_ref, i_out_ref, bufs, sems):
    n = x_hbm.shape[0]
    nchunks = n // CHUNK
    wb = w_ref[...]
    bb = b_ref[...]

    def copy_in(k):
        return pltpu.make_async_copy(
            x_hbm.at[pl.ds(k * CHUNK, CHUNK), :], bufs.at[k % NBUF], sems.at[k % NBUF])

    for k in range(NBUF):
        copy_in(k).start()
    for k in range(nchunks):
        copy_in(k).wait()
        xb = bufs[k % NBUF]
        logits = jax.lax.dot_general(
            xb, wb, (((1,), (1,)), ((), ())),
            preferred_element_type=jnp.float32) + bb
        w2, i2 = _top2_softmax(logits, CHUNK)
        w_out_ref[pl.ds(k * CHUNK, CHUNK), :] = w2
        i_out_ref[pl.ds(k * CHUNK, CHUNK), :] = i2
        if k + NBUF < nchunks:
            copy_in(k + NBUF).start()


@functools.partial(jax.jit, static_argnames=())
def kernel(x, W, b):
    d = x.shape[-1]
    xt = x.reshape(-1, d)
    n = xt.shape[0]

    weights, indices = pl.pallas_call(
        _router_body,
        in_specs=[
            pl.BlockSpec(memory_space=pl.ANY),
            pl.BlockSpec((NUM_EXPERTS, d), lambda: (0, 0)),
            pl.BlockSpec((1, NUM_EXPERTS), lambda: (0, 0)),
        ],
        out_specs=[
            pl.BlockSpec((n, 2), lambda: (0, 0)),
            pl.BlockSpec((n, 2), lambda: (0, 0)),
        ],
        out_shape=[
            jax.ShapeDtypeStruct((n, 2), jnp.float32),
            jax.ShapeDtypeStruct((n, 2), jnp.int32),
        ],
        scratch_shapes=[
            pltpu.VMEM((NBUF, CHUNK, d), jnp.float32),
            pltpu.SemaphoreType.DMA((NBUF,)),
        ],
    )(xt, W, b.reshape(1, NUM_EXPERTS))
    return (weights, indices)


# speedup vs baseline: 1.1108x; 1.0584x over previous
import functools

import jax
import jax.numpy as jnp
from jax import lax
from jax.experimental import pallas as pl
from jax.experimental.pallas import tpu as pltpu

TOKENS_PER_BLOCK = 2048
NUM_EXPERTS = 64
N_TOKENS = 16384
EMB = 2048


def _make_kernel(n, d):
    mesh = pltpu.create_tensorcore_mesh("core")
    t = TOKENS_PER_BLOCK
    nblocks = n // t

    @functools.partial(
        pl.kernel,
        out_type=[
            jax.ShapeDtypeStruct((n, 2), jnp.float32),
            jax.ShapeDtypeStruct((n, 2), jnp.int32),
        ],
        mesh=mesh,
        scratch_types=[
            pltpu.VMEM((NUM_EXPERTS, d), jnp.float32),
            pltpu.VMEM((1, NUM_EXPERTS), jnp.float32),
            pltpu.SemaphoreType.DMA,
        ],
    )
    def run(x_hbm, w_hbm, b_hbm, w_out_hbm, i_out_hbm, w_vmem, b_vmem, sem):
        ncores = lax.axis_size("core")
        cid = lax.axis_index("core")
        bpc = nblocks // ncores

        pltpu.async_copy(w_hbm, w_vmem, sem).wait()
        pltpu.async_copy(b_hbm, b_vmem, sem).wait()

        def block(x_ref, w_out_ref, i_out_ref):
            xb = x_ref[...]
            wb = w_vmem[...]
            logits = jax.lax.dot_general(
                xb, wb, (((1,), (1,)), ((), ())),
                preferred_element_type=jnp.float32)
            logits = logits + b_vmem[...]

            eidx = jax.lax.broadcasted_iota(jnp.int32, (t, NUM_EXPERTS), 1)
            m1 = jnp.max(logits, axis=1, keepdims=True)
            i1 = jnp.argmax(logits, axis=1).reshape(t, 1).astype(jnp.int32)
            masked = jnp.where(eidx == i1, -jnp.inf, logits)
            m2 = jnp.max(masked, axis=1, keepdims=True)
            i2 = jnp.argmax(masked, axis=1).reshape(t, 1).astype(jnp.int32)

            s = jnp.exp(m2 - m1)
            w1 = 1.0 / (1.0 + s)
            w2 = s / (1.0 + s)
            w_out_ref[...] = jnp.concatenate([w1, w2], axis=1)
            i_out_ref[...] = jnp.concatenate([i1, i2], axis=1)

        pltpu.emit_pipeline(
            block,
            grid=(bpc,),
            in_specs=[pl.BlockSpec((t, d), lambda i: (i + cid * bpc, 0))],
            out_specs=[
                pl.BlockSpec((t, 2), lambda i: (i + cid * bpc, 0)),
                pl.BlockSpec((t, 2), lambda i: (i + cid * bpc, 0)),
            ],
        )(x_hbm, w_out_hbm, i_out_hbm)

    return run


@functools.partial(jax.jit, static_argnames=())
def kernel(x, W, b):
    d = x.shape[-1]
    xt = x.reshape(-1, d)
    n = xt.shape[0]
    weights, indices = _make_kernel(n, d)(xt, W, b.reshape(1, NUM_EXPERTS))
    return (weights, indices)


# speedup vs baseline: 1.1211x; 1.0093x over previous
import functools

import jax
import jax.numpy as jnp
from jax.experimental import pallas as pl
from jax.experimental.pallas import tpu as pltpu

TOKENS_PER_BLOCK = 2048
NUM_EXPERTS = 64


def _router_block(x_ref, w_ref, b_ref, w_out_ref, i_out_ref):
    xb = x_ref[...]
    wb = w_ref[...]
    logits = jax.lax.dot_general(
        wb, xb, (((1,), (1,)), ((), ())),
        preferred_element_type=jnp.float32)
    logits = logits + b_ref[...]

    t = logits.shape[1]
    eidx = jax.lax.broadcasted_iota(jnp.int32, (NUM_EXPERTS, t), 0)
    m1 = jnp.max(logits, axis=0, keepdims=True)
    i1 = jnp.argmax(logits, axis=0).reshape(1, t).astype(jnp.int32)
    masked = jnp.where(eidx == i1, -jnp.inf, logits)
    m2 = jnp.max(masked, axis=0, keepdims=True)
    i2 = jnp.argmax(masked, axis=0).reshape(1, t).astype(jnp.int32)

    s = jnp.exp(m2 - m1)
    w1 = 1.0 / (1.0 + s)
    w2 = s / (1.0 + s)

    w_out_ref[...] = jnp.concatenate([w1, w2], axis=0).T
    i_out_ref[...] = jnp.concatenate([i1, i2], axis=0).T


@functools.partial(jax.jit, static_argnames=())
def kernel(x, W, b):
    d = x.shape[-1]
    xt = x.reshape(-1, d)
    n = xt.shape[0]
    t = TOKENS_PER_BLOCK
    grid = (n // t,)

    weights, indices = pl.pallas_call(
        _router_block,
        grid=grid,
        in_specs=[
            pl.BlockSpec((t, d), lambda i: (i, 0)),
            pl.BlockSpec((NUM_EXPERTS, d), lambda i: (0, 0)),
            pl.BlockSpec((NUM_EXPERTS, 1), lambda i: (0, 0)),
        ],
        out_specs=[
            pl.BlockSpec((t, 2), lambda i: (i, 0)),
            pl.BlockSpec((t, 2), lambda i: (i, 0)),
        ],
        out_shape=[
            jax.ShapeDtypeStruct((n, 2), jnp.float32),
            jax.ShapeDtypeStruct((n, 2), jnp.int32),
        ],
        compiler_params=pltpu.CompilerParams(
            dimension_semantics=("parallel",),
        ),
    )(xt, W, b.reshape(NUM_EXPERTS, 1))
    return (weights, indices)


# speedup vs baseline: 1.1381x; 1.0152x over previous
import functools

import jax
import jax.numpy as jnp
from jax.experimental import pallas as pl
from jax.experimental.pallas import tpu as pltpu

TOKENS_PER_BLOCK = 2048
NUM_EXPERTS = 64


def _router_block(x_ref, w_ref, b_ref, w_out_ref, i_out_ref):
    xb = x_ref[...]
    wb = w_ref[...]
    logits = jax.lax.dot_general(
        xb, wb, (((1,), (1,)), ((), ())),
        preferred_element_type=jnp.float32)
    logits = logits + b_ref[...]

    t = logits.shape[0]
    eidx = jax.lax.broadcasted_iota(jnp.int32, (t, NUM_EXPERTS), 1)
    m1 = jnp.max(logits, axis=1, keepdims=True)
    i1 = jnp.argmax(logits, axis=1).reshape(t, 1).astype(jnp.int32)
    masked = jnp.where(eidx == i1, -jnp.inf, logits)
    m2 = jnp.max(masked, axis=1, keepdims=True)
    i2 = jnp.argmax(masked, axis=1).reshape(t, 1).astype(jnp.int32)

    s = jnp.exp(m2 - m1)
    w1 = 1.0 / (1.0 + s)
    w2 = s / (1.0 + s)

    w_out_ref[...] = jnp.concatenate([w1, w2], axis=1)
    i_out_ref[...] = jnp.concatenate([i1, i2], axis=1)


@functools.partial(jax.jit, static_argnames=())
def kernel(x, W, b):
    d = x.shape[-1]
    xt = x.reshape(-1, d)
    n = xt.shape[0]
    t = TOKENS_PER_BLOCK
    grid = (n // t,)

    weights, indices = pl.pallas_call(
        _router_block,
        grid=grid,
        in_specs=[
            pl.BlockSpec((t, d), lambda i: (i, 0)),
            pl.BlockSpec((NUM_EXPERTS, d), lambda i: (0, 0)),
            pl.BlockSpec((1, NUM_EXPERTS), lambda i: (0, 0)),
        ],
        out_specs=[
            pl.BlockSpec((t, 2), lambda i: (i, 0)),
            pl.BlockSpec((t, 2), lambda i: (i, 0)),
        ],
        out_shape=[
            jax.ShapeDtypeStruct((n, 2), jnp.float32),
            jax.ShapeDtypeStruct((n, 2), jnp.int32),
        ],
        compiler_params=pltpu.CompilerParams(
            dimension_semantics=("parallel",),
        ),
    )(xt, W, b.reshape(1, NUM_EXPERTS))
    return (weights, indices)
